# baseline (speedup 1.0000x reference)
import numpy as np
import jax
import jax.numpy as jnp
from jax.experimental import pallas as pl
from jax.experimental.pallas import tpu as pltpu

_SIGMA = 0.2
_BATCH = 64
_H, _W, _C = 56, 56, 192
_N = _H * _W * _C
_LANES = 128
_SUB = _N // _LANES

_PERM = [17, 27, 42, 32, 1, 3, 58, 51, 40, 28, 52, 19, 9, 33, 11, 45,
         31, 5, 15, 39, 50, 47, 20, 0, 46, 14, 49, 44, 38, 61, 2, 54,
         36, 35, 62, 63, 21, 59, 30, 43, 22, 18, 24, 26, 53, 12, 16, 6,
         7, 57, 55, 48, 13, 37, 60, 10, 29, 34, 25, 56, 4, 41, 23, 8]


def _cycle_schedule():
    seen = [False] * _BATCH
    in_rows, out_rows = [], []
    for start in range(_BATCH):
        if seen[start]:
            continue
        cyc = []
        i = start
        while not seen[i]:
            seen[i] = True
            cyc.append(i)
            i = _PERM[i]
        for k, row in enumerate(cyc):
            nxt = cyc[(k + 1) % len(cyc)]
            in_rows.extend((row, nxt))
            out_rows.extend((row, row))
    return np.asarray([in_rows, out_rows], dtype=np.int32)


_SCHED = _cycle_schedule()
_STEPS = _SCHED.shape[1]


def _mix_body(sched_ref, x_ref, a_ref, b_ref, o_ref):
    phase = pl.program_id(0) % 2

    @pl.when(phase == 0)
    def _():
        o_ref[...] = x_ref[...] * a_ref[...]

    @pl.when(phase == 1)
    def _():
        o_ref[...] += x_ref[...] * b_ref[...]


def _coeffs():
    key = jax.random.key(42)
    _, k_r, k_theta = jax.random.split(key, 3)
    rs = (1, _H, _W, _C)
    r = jax.random.normal(k_r, rs, dtype=jnp.float16) * jnp.float16(_SIGMA)
    theta = jax.random.uniform(
        k_theta, rs, dtype=jnp.float16, minval=-np.pi, maxval=np.pi)
    a = (jnp.float16(1.0) + r * jnp.cos(theta)).astype(jnp.float32)
    b = (r * jnp.sin(theta)).astype(jnp.float32)
    return a.reshape(_SUB, _LANES), b.reshape(_SUB, _LANES)


def kernel(inputs):
    x = inputs.reshape(_BATCH, _SUB, _LANES)
    a, b = _coeffs()
    sched = jnp.asarray(_SCHED)
    grid_spec = pltpu.PrefetchScalarGridSpec(
        num_scalar_prefetch=1,
        grid=(_STEPS,),
        in_specs=[
            pl.BlockSpec((1, _SUB, _LANES), lambda i, s: (s[0, i], 0, 0)),
            pl.BlockSpec((_SUB, _LANES), lambda i, s: (0, 0)),
            pl.BlockSpec((_SUB, _LANES), lambda i, s: (0, 0)),
        ],
        out_specs=pl.BlockSpec((1, _SUB, _LANES), lambda i, s: (s[1, i], 0, 0)),
    )
    y = pl.pallas_call(
        _mix_body,
        grid_spec=grid_spec,
        out_shape=jax.ShapeDtypeStruct((_BATCH, _SUB, _LANES), jnp.float32),
    )(sched, x, a, b)
    return y.reshape(inputs.shape)


# speedup vs baseline: 1.0825x; 1.0825x over previous
import numpy as np
import jax
import jax.numpy as jnp
from jax.experimental import pallas as pl
from jax.experimental.pallas import tpu as pltpu

_SIGMA = 0.2
_BATCH = 64
_H, _W, _C = 56, 56, 192
_N = _H * _W * _C
_LANES = 128
_SUB = _N // _LANES

_PERM = [17, 27, 42, 32, 1, 3, 58, 51, 40, 28, 52, 19, 9, 33, 11, 45,
         31, 5, 15, 39, 50, 47, 20, 0, 46, 14, 49, 44, 38, 61, 2, 54,
         36, 35, 62, 63, 21, 59, 30, 43, 22, 18, 24, 26, 53, 12, 16, 6,
         7, 57, 55, 48, 13, 37, 60, 10, 29, 34, 25, 56, 4, 41, 23, 8]


def _cycle_schedule():
    seen = [False] * _BATCH
    in_rows, out_rows = [], []
    for start in range(_BATCH):
        if seen[start]:
            continue
        cyc = []
        i = start
        while not seen[i]:
            seen[i] = True
            cyc.append(i)
            i = _PERM[i]
        for k, row in enumerate(cyc):
            nxt = cyc[(k + 1) % len(cyc)]
            in_rows.extend((row, nxt))
            out_rows.extend((row, row))
    return np.asarray([in_rows, out_rows], dtype=np.int32)


_SCHED = _cycle_schedule()
_STEPS = _SCHED.shape[1]


def _mix_body(perm_ref, xc_ref, xp0_ref, xp1_ref, a_ref, b_ref, o_ref):
    o_ref[0] = xc_ref[0] * a_ref[...] + xp0_ref[0] * b_ref[...]
    o_ref[1] = xc_ref[1] * a_ref[...] + xp1_ref[0] * b_ref[...]


def _coeffs():
    key = jax.random.key(42)
    _, k_r, k_theta = jax.random.split(key, 3)
    rs = (1, _H, _W, _C)
    r = jax.random.normal(k_r, rs, dtype=jnp.float16) * jnp.float16(_SIGMA)
    theta = jax.random.uniform(
        k_theta, rs, dtype=jnp.float16, minval=-np.pi, maxval=np.pi)
    a = (jnp.float16(1.0) + r * jnp.cos(theta)).astype(jnp.float32)
    b = (r * jnp.sin(theta)).astype(jnp.float32)
    return a.reshape(_SUB, _LANES), b.reshape(_SUB, _LANES)


def kernel(inputs):
    x = inputs.reshape(_BATCH, _SUB, _LANES)
    a, b = _coeffs()
    perm = jnp.asarray(np.asarray(_PERM, dtype=np.int32))
    grid_spec = pltpu.PrefetchScalarGridSpec(
        num_scalar_prefetch=1,
        grid=(_BATCH // 2,),
        in_specs=[
            pl.BlockSpec((2, _SUB, _LANES), lambda i, p: (i, 0, 0)),
            pl.BlockSpec((1, _SUB, _LANES), lambda i, p: (p[2 * i], 0, 0)),
            pl.BlockSpec((1, _SUB, _LANES), lambda i, p: (p[2 * i + 1], 0, 0)),
            pl.BlockSpec((_SUB, _LANES), lambda i, p: (0, 0)),
            pl.BlockSpec((_SUB, _LANES), lambda i, p: (0, 0)),
        ],
        out_specs=pl.BlockSpec((2, _SUB, _LANES), lambda i, p: (i, 0, 0)),
    )
    y = pl.pallas_call(
        _mix_body,
        grid_spec=grid_spec,
        out_shape=jax.ShapeDtypeStruct((_BATCH, _SUB, _LANES), jnp.float32),
    )(perm, x, x, x, a, b)
    return y.reshape(inputs.shape)


# speedup vs baseline: 1.1618x; 1.0732x over previous
import numpy as np
import jax
import jax.numpy as jnp
from jax.experimental import pallas as pl
from jax.experimental.pallas import tpu as pltpu

_SIGMA = 0.2
_BATCH = 64
_H, _W, _C = 56, 56, 192
_N = _H * _W * _C
_LANES = 128
_SUB = _N // _LANES

_NBUF = 8
_DEPTH = 6
_NOBUF = 4

_PERM = [17, 27, 42, 32, 1, 3, 58, 51, 40, 28, 52, 19, 9, 33, 11, 45,
         31, 5, 15, 39, 50, 47, 20, 0, 46, 14, 49, 44, 38, 61, 2, 54,
         36, 35, 62, 63, 21, 59, 30, 43, 22, 18, 24, 26, 53, 12, 16, 6,
         7, 57, 55, 48, 13, 37, 60, 10, 29, 34, 25, 56, 4, 41, 23, 8]


def _schedule():
    seen = [False] * _BATCH
    fetches, outputs = [], []
    for start in range(_BATCH):
        if seen[start]:
            continue
        cyc = []
        i = start
        while not seen[i]:
            seen[i] = True
            cyc.append(i)
            i = _PERM[i]
        base = len(fetches)
        fetches.extend(cyc)
        fetches.append(cyc[0])
        for k, row in enumerate(cyc):
            outputs.append((row, base + k, base + k + 1))

    nf, nk = len(fetches), len(outputs)
    out_row = np.asarray([o[0] for o in outputs], np.int32)
    fa = np.asarray([o[1] for o in outputs], np.int32)
    fb = np.asarray([o[2] for o in outputs], np.int32)
    f_hi = np.minimum(fb + _DEPTH + 1, nf).astype(np.int32)
    f_lo = np.empty_like(f_hi)
    f_lo[0] = 0
    f_lo[1:] = f_hi[:-1]
    w_hi = (fb + 1).astype(np.int32)
    w_lo = np.empty_like(w_hi)
    w_lo[0] = 0
    w_lo[1:] = w_hi[:-1]
    return (np.asarray(fetches, np.int32), out_row, fa, fb,
            f_lo, f_hi, w_lo, w_hi, nk)


(_FETCH_ROW, _OUT_ROW, _FA, _FB, _FLO, _FHI, _WLO, _WHI, _STEPS) = _schedule()


def _mix_body(fetch_ref, orow_ref, fa_ref, fb_ref, flo_ref, fhi_ref,
              wlo_ref, whi_ref, x_hbm, a_ref, b_ref, o_hbm,
              ibufs, obufs, isems, osems):
    k = pl.program_id(0)

    def _in_copy(f):
        row = fetch_ref[f]
        slot = jax.lax.rem(f, _NBUF)
        return pltpu.make_async_copy(
            x_hbm.at[row], ibufs.at[slot], isems.at[slot])

    def _out_copy(step):
        row = orow_ref[step]
        slot = jax.lax.rem(step, _NOBUF)
        return pltpu.make_async_copy(
            obufs.at[slot], o_hbm.at[row], osems.at[slot])

    def _issue(f, carry):
        _in_copy(f).start()
        return carry

    jax.lax.fori_loop(flo_ref[k], fhi_ref[k], _issue, 0)

    def _wait(f, carry):
        _in_copy(f).wait()
        return carry

    jax.lax.fori_loop(wlo_ref[k], whi_ref[k], _wait, 0)

    oslot = jax.lax.rem(k, _NOBUF)

    @pl.when(k >= _NOBUF)
    def _():
        _out_copy(k - _NOBUF).wait()

    fa_slot = jax.lax.rem(fa_ref[k], _NBUF)
    fb_slot = jax.lax.rem(fb_ref[k], _NBUF)
    obufs[oslot] = ibufs[fa_slot] * a_ref[...] + ibufs[fb_slot] * b_ref[...]

    _out_copy(k).start()

    @pl.when(k == _STEPS - 1)
    def _():
        for step in range(_STEPS - _NOBUF, _STEPS):
            _out_copy(step).wait()


def _coeffs():
    key = jax.random.key(42)
    _, k_r, k_theta = jax.random.split(key, 3)
    rs = (1, _H, _W, _C)
    r = jax.random.normal(k_r, rs, dtype=jnp.float16) * jnp.float16(_SIGMA)
    theta = jax.random.uniform(
        k_theta, rs, dtype=jnp.float16, minval=-np.pi, maxval=np.pi)
    a = (jnp.float16(1.0) + r * jnp.cos(theta)).astype(jnp.float32)
    b = (r * jnp.sin(theta)).astype(jnp.float32)
    return a.reshape(_SUB, _LANES), b.reshape(_SUB, _LANES)


def kernel(inputs):
    x = inputs.reshape(_BATCH, _SUB, _LANES)
    a, b = _coeffs()
    grid_spec = pltpu.PrefetchScalarGridSpec(
        num_scalar_prefetch=8,
        grid=(_STEPS,),
        in_specs=[
            pl.BlockSpec(memory_space=pltpu.HBM),
            pl.BlockSpec(memory_space=pltpu.VMEM),
            pl.BlockSpec(memory_space=pltpu.VMEM),
        ],
        out_specs=pl.BlockSpec(memory_space=pltpu.HBM),
        scratch_shapes=[
            pltpu.VMEM((_NBUF, _SUB, _LANES), jnp.float32),
            pltpu.VMEM((_NOBUF, _SUB, _LANES), jnp.float32),
            pltpu.SemaphoreType.DMA((_NBUF,)),
            pltpu.SemaphoreType.DMA((_NOBUF,)),
        ],
    )
    y = pl.pallas_call(
        _mix_body,
        grid_spec=grid_spec,
        out_shape=jax.ShapeDtypeStruct((_BATCH, _SUB, _LANES), jnp.float32),
    )(jnp.asarray(_FETCH_ROW), jnp.asarray(_OUT_ROW), jnp.asarray(_FA),
      jnp.asarray(_FB), jnp.asarray(_FLO), jnp.asarray(_FHI),
      jnp.asarray(_WLO), jnp.asarray(_WHI), x, a, b)
    return y.reshape(inputs.shape)


# speedup vs baseline: 1.1643x; 1.0022x over previous
import numpy as np
import jax
import jax.numpy as jnp
from jax.experimental import pallas as pl
from jax.experimental.pallas import tpu as pltpu

_SIGMA = 0.2
_BATCH = 64
_H, _W, _C = 56, 56, 192
_N = _H * _W * _C
_LANES = 128
_SUB = _N // _LANES

_NBUF = 8
_DEPTH = 6
_NOBUF = 4

_PERM = [17, 27, 42, 32, 1, 3, 58, 51, 40, 28, 52, 19, 9, 33, 11, 45,
         31, 5, 15, 39, 50, 47, 20, 0, 46, 14, 49, 44, 38, 61, 2, 54,
         36, 35, 62, 63, 21, 59, 30, 43, 22, 18, 24, 26, 53, 12, 16, 6,
         7, 57, 55, 48, 13, 37, 60, 10, 29, 34, 25, 56, 4, 41, 23, 8]


def _schedule():
    seen = [False] * _BATCH
    fetches, outputs = [], []
    for start in range(_BATCH):
        if seen[start]:
            continue
        cyc = []
        i = start
        while not seen[i]:
            seen[i] = True
            cyc.append(i)
            i = _PERM[i]
        base = len(fetches)
        fetches.extend(cyc)
        fetches.append(cyc[0])
        for k, row in enumerate(cyc):
            outputs.append((row, base + k, base + k + 1))

    nf, nk = len(fetches), len(outputs)
    out_row = np.asarray([o[0] for o in outputs], np.int32)
    fa = np.asarray([o[1] for o in outputs], np.int32)
    fb = np.asarray([o[2] for o in outputs], np.int32)
    f_hi = np.minimum(fb + _DEPTH + 1, nf).astype(np.int32)
    f_lo = np.empty_like(f_hi)
    f_lo[0] = 0
    f_lo[1:] = f_hi[:-1]
    w_hi = (fb + 1).astype(np.int32)
    w_lo = np.empty_like(w_hi)
    w_lo[0] = 0
    w_lo[1:] = w_hi[:-1]
    return (np.asarray(fetches, np.int32), out_row, fa, fb,
            f_lo, f_hi, w_lo, w_hi, nk)


(_FETCH_ROW, _OUT_ROW, _FA, _FB, _FLO, _FHI, _WLO, _WHI, _STEPS) = _schedule()


def _mix_body(fetch_ref, orow_ref, fa_ref, fb_ref, flo_ref, fhi_ref,
              wlo_ref, whi_ref, x_hbm, a_ref, b_ref, o_hbm,
              ibufs, obufs, isems, osems):
    k = pl.program_id(0)

    def _in_copy(f):
        row = fetch_ref[f]
        slot = jax.lax.rem(f, _NBUF)
        return pltpu.make_async_copy(
            x_hbm.at[row], ibufs.at[slot], isems.at[slot])

    def _out_copy(step):
        row = orow_ref[step]
        slot = jax.lax.rem(step, _NOBUF)
        return pltpu.make_async_copy(
            obufs.at[slot], o_hbm.at[row], osems.at[slot])

    def _issue(f, carry):
        cp = _in_copy(f)
        lane = jax.lax.rem(f, 2)
        for j in range(2):
            @pl.when(lane == j)
            def _(cp=cp, j=j):
                cp.start(priority=j)
        return carry

    jax.lax.fori_loop(flo_ref[k], fhi_ref[k], _issue, 0)

    def _wait(f, carry):
        _in_copy(f).wait()
        return carry

    jax.lax.fori_loop(wlo_ref[k], whi_ref[k], _wait, 0)

    oslot = jax.lax.rem(k, _NOBUF)

    @pl.when(k >= _NOBUF)
    def _():
        _out_copy(k - _NOBUF).wait()

    fa_slot = jax.lax.rem(fa_ref[k], _NBUF)
    fb_slot = jax.lax.rem(fb_ref[k], _NBUF)
    obufs[oslot] = ibufs[fa_slot] * a_ref[...] + ibufs[fb_slot] * b_ref[...]

    ocp = _out_copy(k)
    olane = jax.lax.rem(k, 2)
    for j in range(2):
        @pl.when(olane == j)
        def _(ocp=ocp, j=j):
            ocp.start(priority=j)

    @pl.when(k == _STEPS - 1)
    def _():
        for step in range(_STEPS - _NOBUF, _STEPS):
            _out_copy(step).wait()


def _coeffs():
    key = jax.random.key(42)
    _, k_r, k_theta = jax.random.split(key, 3)
    rs = (1, _H, _W, _C)
    r = jax.random.normal(k_r, rs, dtype=jnp.float16) * jnp.float16(_SIGMA)
    theta = jax.random.uniform(
        k_theta, rs, dtype=jnp.float16, minval=-np.pi, maxval=np.pi)
    a = (jnp.float16(1.0) + r * jnp.cos(theta)).astype(jnp.float32)
    b = (r * jnp.sin(theta)).astype(jnp.float32)
    return a.reshape(_SUB, _LANES), b.reshape(_SUB, _LANES)


def kernel(inputs):
    x = inputs.reshape(_BATCH, _SUB, _LANES)
    a, b = _coeffs()
    grid_spec = pltpu.PrefetchScalarGridSpec(
        num_scalar_prefetch=8,
        grid=(_STEPS,),
        in_specs=[
            pl.BlockSpec(memory_space=pltpu.HBM),
            pl.BlockSpec(memory_space=pltpu.VMEM),
            pl.BlockSpec(memory_space=pltpu.VMEM),
        ],
        out_specs=pl.BlockSpec(memory_space=pltpu.HBM),
        scratch_shapes=[
            pltpu.VMEM((_NBUF, _SUB, _LANES), jnp.float32),
            pltpu.VMEM((_NOBUF, _SUB, _LANES), jnp.float32),
            pltpu.SemaphoreType.DMA((_NBUF,)),
            pltpu.SemaphoreType.DMA((_NOBUF,)),
        ],
    )
    y = pl.pallas_call(
        _mix_body,
        grid_spec=grid_spec,
        out_shape=jax.ShapeDtypeStruct((_BATCH, _SUB, _LANES), jnp.float32),
    )(jnp.asarray(_FETCH_ROW), jnp.asarray(_OUT_ROW), jnp.asarray(_FA),
      jnp.asarray(_FB), jnp.asarray(_FLO), jnp.asarray(_FHI),
      jnp.asarray(_WLO), jnp.asarray(_WHI), x, a, b)
    return y.reshape(inputs.shape)


# speedup vs baseline: 1.1648x; 1.0004x over previous
import numpy as np
import jax
import jax.numpy as jnp
from jax.experimental import pallas as pl
from jax.experimental.pallas import tpu as pltpu

_SIGMA = 0.2
_BATCH = 64
_H, _W, _C = 56, 56, 192
_N = _H * _W * _C
_LANES = 128
_SUB = _N // _LANES
_CHUNK = 336
_NCHUNK = _SUB // _CHUNK

_PERM = [17, 27, 42, 32, 1, 3, 58, 51, 40, 28, 52, 19, 9, 33, 11, 45,
         31, 5, 15, 39, 50, 47, 20, 0, 46, 14, 49, 44, 38, 61, 2, 54,
         36, 35, 62, 63, 21, 59, 30, 43, 22, 18, 24, 26, 53, 12, 16, 6,
         7, 57, 55, 48, 13, 37, 60, 10, 29, 34, 25, 56, 4, 41, 23, 8]


def _mix_body(x_ref, a_ref, b_ref, o_ref):
    for i in range(_BATCH):
        o_ref[i] = x_ref[i] * a_ref[...] + x_ref[_PERM[i]] * b_ref[...]


def _coeffs():
    key = jax.random.key(42)
    _, k_r, k_theta = jax.random.split(key, 3)
    rs = (1, _H, _W, _C)
    r = jax.random.normal(k_r, rs, dtype=jnp.float16) * jnp.float16(_SIGMA)
    theta = jax.random.uniform(
        k_theta, rs, dtype=jnp.float16, minval=-np.pi, maxval=np.pi)
    a = (jnp.float16(1.0) + r * jnp.cos(theta)).astype(jnp.float32)
    b = (r * jnp.sin(theta)).astype(jnp.float32)
    return a.reshape(_SUB, _LANES), b.reshape(_SUB, _LANES)


def kernel(inputs):
    x = inputs.reshape(_BATCH, _SUB, _LANES)
    a, b = _coeffs()
    y = pl.pallas_call(
        _mix_body,
        grid=(_NCHUNK,),
        in_specs=[
            pl.BlockSpec((_BATCH, _CHUNK, _LANES), lambda c: (0, c, 0)),
            pl.BlockSpec((_CHUNK, _LANES), lambda c: (c, 0)),
            pl.BlockSpec((_CHUNK, _LANES), lambda c: (c, 0)),
        ],
        out_specs=pl.BlockSpec((_BATCH, _CHUNK, _LANES), lambda c: (0, c, 0)),
        out_shape=jax.ShapeDtypeStruct((_BATCH, _SUB, _LANES), jnp.float32),
    )(x, a, b)
    return y.reshape(inputs.shape)


# speedup vs baseline: 3.1960x; 2.7438x over previous
import numpy as np
import jax
import jax.numpy as jnp
from jax.experimental import pallas as pl
from jax.experimental.pallas import tpu as pltpu

_SIGMA = 0.2
_BATCH = 64
_H, _W, _C = 56, 56, 192
_HW = _H * _W
_BH = 112
_NCHUNK = _HW // _BH

_PERM = [17, 27, 42, 32, 1, 3, 58, 51, 40, 28, 52, 19, 9, 33, 11, 45,
         31, 5, 15, 39, 50, 47, 20, 0, 46, 14, 49, 44, 38, 61, 2, 54,
         36, 35, 62, 63, 21, 59, 30, 43, 22, 18, 24, 26, 53, 12, 16, 6,
         7, 57, 55, 48, 13, 37, 60, 10, 29, 34, 25, 56, 4, 41, 23, 8]


def _mix_body(x_ref, a_ref, b_ref, o_ref):
    for i in range(_BATCH):
        o_ref[i] = x_ref[i] * a_ref[...] + x_ref[_PERM[i]] * b_ref[...]


def _coeffs():
    key = jax.random.key(42)
    _, k_r, k_theta = jax.random.split(key, 3)
    rs = (1, _H, _W, _C)
    r = jax.random.normal(k_r, rs, dtype=jnp.float16) * jnp.float16(_SIGMA)
    theta = jax.random.uniform(
        k_theta, rs, dtype=jnp.float16, minval=-np.pi, maxval=np.pi)
    a = (jnp.float16(1.0) + r * jnp.cos(theta)).astype(jnp.float32)
    b = (r * jnp.sin(theta)).astype(jnp.float32)
    return a.reshape(_HW, _C), b.reshape(_HW, _C)


def kernel(inputs):
    x = inputs.reshape(_BATCH, _HW, _C)
    a, b = _coeffs()
    y = pl.pallas_call(
        _mix_body,
        grid=(_NCHUNK,),
        in_specs=[
            pl.BlockSpec((_BATCH, _BH, _C), lambda c: (0, c, 0)),
            pl.BlockSpec((_BH, _C), lambda c: (c, 0)),
            pl.BlockSpec((_BH, _C), lambda c: (c, 0)),
        ],
        out_specs=pl.BlockSpec((_BATCH, _BH, _C), lambda c: (0, c, 0)),
        out_shape=jax.ShapeDtypeStruct((_BATCH, _HW, _C), jnp.float32),
    )(x, a, b)
    return y.reshape(inputs.shape)


# speedup vs baseline: 3.8297x; 1.1983x over previous
import numpy as np
import jax
import jax.numpy as jnp
from jax.experimental import pallas as pl
from jax.experimental.pallas import tpu as pltpu

_SIGMA = 0.2
_BATCH = 64
_H, _W, _C = 56, 56, 192
_HW = _H * _W
_BH = 112
_NCHUNK = _HW // _BH

_PERM = [17, 27, 42, 32, 1, 3, 58, 51, 40, 28, 52, 19, 9, 33, 11, 45,
         31, 5, 15, 39, 50, 47, 20, 0, 46, 14, 49, 44, 38, 61, 2, 54,
         36, 35, 62, 63, 21, 59, 30, 43, 22, 18, 24, 26, 53, 12, 16, 6,
         7, 57, 55, 48, 13, 37, 60, 10, 29, 34, 25, 56, 4, 41, 23, 8]


def _mix_body(x_ref, r_ref, t_ref, o_ref):
    r = r_ref[...]
    t = t_ref[...]
    a = 1.0 + r * jnp.cos(t)
    b = r * jnp.sin(t)
    for i in range(_BATCH):
        o_ref[i] = x_ref[i] * a + x_ref[_PERM[i]] * b


def _draws():
    key = jax.random.key(42)
    _, k_r, k_theta = jax.random.split(key, 3)
    r = jax.random.normal(k_r, (_HW, _C), dtype=jnp.float16) \
        * jnp.float16(_SIGMA)
    theta = jax.random.uniform(
        k_theta, (_HW, _C), dtype=jnp.float16, minval=-np.pi, maxval=np.pi)
    return r.astype(jnp.float32), theta.astype(jnp.float32)


def kernel(inputs):
    x = inputs.reshape(_BATCH, _HW, _C)
    r, theta = _draws()
    y = pl.pallas_call(
        _mix_body,
        grid=(_NCHUNK,),
        in_specs=[
            pl.BlockSpec((_BATCH, _BH, _C), lambda c: (0, c, 0)),
            pl.BlockSpec((_BH, _C), lambda c: (c, 0)),
            pl.BlockSpec((_BH, _C), lambda c: (c, 0)),
        ],
        out_specs=pl.BlockSpec((_BATCH, _BH, _C), lambda c: (0, c, 0)),
        out_shape=jax.ShapeDtypeStruct((_BATCH, _HW, _C), jnp.float32),
    )(x, r, theta)
    return y.reshape(inputs.shape)


# speedup vs baseline: 3.9828x; 1.0400x over previous
import numpy as np
import jax
import jax.numpy as jnp
from jax.experimental import pallas as pl
from jax.experimental.pallas import tpu as pltpu

_SIGMA = 0.2
_BATCH = 64
_H, _W, _C = 56, 56, 192
_HW = _H * _W
_BH = 112
_NCHUNK = _HW // _BH

_PERM = [17, 27, 42, 32, 1, 3, 58, 51, 40, 28, 52, 19, 9, 33, 11, 45,
         31, 5, 15, 39, 50, 47, 20, 0, 46, 14, 49, 44, 38, 61, 2, 54,
         36, 35, 62, 63, 21, 59, 30, 43, 22, 18, 24, 26, 53, 12, 16, 6,
         7, 57, 55, 48, 13, 37, 60, 10, 29, 34, 25, 56, 4, 41, 23, 8]


def _mix_body(x_ref, r_ref, t_ref, o_ref):
    r = r_ref[...]
    t = t_ref[...]
    a = 1.0 + r * jnp.cos(t)
    b = r * jnp.sin(t)
    for i in range(_BATCH):
        o_ref[i] = x_ref[i] * a + x_ref[_PERM[i]] * b


def _draws():
    key = jax.random.key(42)
    _, k_r, k_theta = jax.random.split(key, 3)
    rs = (_HW * _C // 128, 128)
    r = jax.random.normal(k_r, rs, dtype=jnp.float16) * jnp.float16(_SIGMA)
    theta = jax.random.uniform(
        k_theta, rs, dtype=jnp.float16, minval=-np.pi, maxval=np.pi)
    return (r.astype(jnp.float32).reshape(_HW, _C),
            theta.astype(jnp.float32).reshape(_HW, _C))


def kernel(inputs):
    x = inputs.reshape(_BATCH, _HW, _C)
    r, theta = _draws()
    y = pl.pallas_call(
        _mix_body,
        grid=(_NCHUNK,),
        in_specs=[
            pl.BlockSpec((_BATCH, _BH, _C), lambda c: (0, c, 0)),
            pl.BlockSpec((_BH, _C), lambda c: (c, 0)),
            pl.BlockSpec((_BH, _C), lambda c: (c, 0)),
        ],
        out_specs=pl.BlockSpec((_BATCH, _BH, _C), lambda c: (0, c, 0)),
        out_shape=jax.ShapeDtypeStruct((_BATCH, _HW, _C), jnp.float32),
    )(x, r, theta)
    return y.reshape(inputs.shape)
